# baseline (speedup 1.0000x reference)
import functools

import jax
import jax.numpy as jnp
from jax import lax
from jax.experimental import pallas as pl
from jax.experimental.pallas import tpu as pltpu
from jax.experimental.pallas import tpu_sc as plsc

N_DOC = 2000
N_NODES = 10000
N_EDGES = 320000
D = 128
NPAD = 10240

NC, NS = 2, 16
NW = NC * NS
EPT = N_EDGES // NW
K = 80
NSTEPS = EPT // K
RPT = NPAD // NS
L = 16
CH = 2000
NBUF = 4


@functools.cache
def _sc_kernels():
    mesh = plsc.VectorSubcoreMesh(core_axis_name="c", subcore_axis_name="s",
                                  num_cores=NC, num_subcores=NS)
    deg = functools.partial(
        pl.kernel,
        out_type=jax.ShapeDtypeStruct((NW, NPAD), jnp.float32),
        mesh=mesh,
        compiler_params=pltpu.CompilerParams(needs_layout_passes=False),
        scratch_types=[
            pltpu.VMEM((CH,), jnp.int32),
            pltpu.VMEM((NPAD,), jnp.float32),
        ],
    )(_deg_body)
    scat = functools.partial(
        pl.kernel,
        out_type=[jax.ShapeDtypeStruct((NPAD, D), jnp.float32),
                  jax.ShapeDtypeStruct((NPAD, D), jnp.float32)],
        mesh=mesh,
        scratch_types=(
            [pltpu.VMEM((K,), jnp.int32)] * NBUF
            + [pltpu.VMEM((K,), jnp.int32)] * NBUF
            + [pltpu.VMEM((K, D), jnp.float32)] * NBUF
            + [pltpu.VMEM_SHARED((NPAD, D), jnp.float32)]
            + [pltpu.SemaphoreType.DMA] * (3 * NBUF)
        ),
    )(_edge_scatter_body)
    return deg, scat


def _deg_body(dst_hbm, out_hbm, didx_v, hist_v):
    c = lax.axis_index("c")
    s = lax.axis_index("s")
    wid = c * NS + s
    base = wid * EPT
    zero = jnp.zeros((L,), jnp.float32)

    @pl.loop(0, NPAD // L)
    def _(i):
        hist_v[pl.ds(i * L, L)] = zero

    ones = jnp.ones((L,), jnp.float32)

    @pl.loop(0, EPT // CH)
    def _(jc):
        pltpu.sync_copy(dst_hbm.at[pl.ds(base + jc * CH, CH)], didx_v)

        @pl.loop(0, CH // L)
        def _(i):
            idx = didx_v[pl.ds(i * L, L)]
            plsc.addupdate_scatter(hist_v, [idx], ones)

    pltpu.sync_copy(hist_v, out_hbm.at[wid])


def _edge_scatter_body(src_hbm, dst_hbm, g_hbm, zerosd_hbm,
                       out0_hbm, out1_hbm, *refs):
    sidx = refs[0:NBUF]
    didx = refs[NBUF:2 * NBUF]
    rows = refs[2 * NBUF:3 * NBUF]
    acc_s = refs[3 * NBUF]
    isem = refs[3 * NBUF + 1:4 * NBUF + 1]
    gsem = refs[4 * NBUF + 1:5 * NBUF + 1]
    ssem = refs[5 * NBUF + 1:6 * NBUF + 1]
    c = lax.axis_index("c")
    s = lax.axis_index("s")
    base = (c * NS + s) * EPT
    r0 = s * RPT

    pltpu.sync_copy(zerosd_hbm.at[pl.ds(r0, RPT)], acc_s.at[pl.ds(r0, RPT)])
    plsc.subcore_barrier()

    def ifire(j, b):
        e0 = base + j * K
        pltpu.async_copy(src_hbm.at[pl.ds(e0, K)], sidx[b], isem[b])
        pltpu.async_copy(dst_hbm.at[pl.ds(e0, K)], didx[b], isem[b])

    def iwait(b):
        pltpu.make_async_copy(src_hbm.at[pl.ds(base, K)], sidx[b],
                              isem[b]).wait()
        pltpu.make_async_copy(dst_hbm.at[pl.ds(base, K)], didx[b],
                              isem[b]).wait()

    def gfire(b):
        pltpu.async_copy(g_hbm.at[sidx[b]], rows[b], gsem[b])

    def gwait(b):
        pltpu.make_async_copy(g_hbm.at[sidx[b]], rows[b], gsem[b]).wait()

    def sfire(b):
        pltpu.async_copy(rows[b], acc_s.at[didx[b]], ssem[b], add=True)

    def swait(b):
        pltpu.make_async_copy(rows[b], acc_s.at[didx[b]], ssem[b]).wait()

    def body(j, m, do_swait=True, do_i=True, do_g=True):
        if do_swait:
            swait((m + NBUF - 1) % NBUF)
        if do_i:
            ifire(j + 3, (m + 3) % NBUF)
        if do_g:
            iwait((m + 2) % NBUF)
            gfire((m + 2) % NBUF)
        gwait(m)
        sfire(m)

    ifire(0, 0)
    ifire(1, 1)
    ifire(2, 2)
    iwait(0)
    gfire(0)
    iwait(1)
    gfire(1)
    body(0, 0, do_swait=False)
    body(1, 1)
    body(2, 2)

    NGRP = (NSTEPS - 3 - 6) // NBUF

    @pl.loop(0, NGRP)
    def _(p):
        j = 3 + NBUF * p
        for t in range(NBUF):
            body(j + t, (3 + t) % NBUF)

    for j in range(3 + NBUF * NGRP, NSTEPS):
        body(j, j % NBUF, do_i=(j + 3 < NSTEPS), do_g=(j + 2 < NSTEPS))
    swait((NSTEPS - 1) % NBUF)

    plsc.subcore_barrier()

    @pl.when(c == 0)
    def _():
        pltpu.sync_copy(acc_s.at[pl.ds(r0, RPT)], out0_hbm.at[pl.ds(r0, RPT)])

    @pl.when(c == 1)
    def _():
        pltpu.sync_copy(acc_s.at[pl.ds(r0, RPT)], out1_hbm.at[pl.ds(r0, RPT)])


BROW = 2000


def _tc1_body(doc_ref, word_ref, linw_ref, linb_ref, w1_ref, hist_ref,
              g1_ref, dinv_ref):
    dinv_full = lax.rsqrt(jnp.sum(hist_ref[...], axis=0)[:, None] + 1.0)
    dinv_ref[...] = dinv_full
    dinv = dinv_full[0:N_NODES]
    wf = jnp.dot(word_ref[...], linw_ref[...],
                 preferred_element_type=jnp.float32) + linb_ref[...]
    hd = jnp.dot(doc_ref[...], w1_ref[...], preferred_element_type=jnp.float32)
    hw = jnp.dot(wf, w1_ref[...], preferred_element_type=jnp.float32)
    g1_ref[0:N_DOC, :] = dinv[0:N_DOC] * hd
    g1_ref[N_DOC:N_NODES, :] = dinv[N_DOC:N_NODES] * hw


def _tc2_body(dinv_ref, acca_ref, accb_ref, g1_ref, b1_ref, w2_ref, g2_ref):
    dinv = dinv_ref[0:N_NODES, :]
    t = acca_ref[0:N_NODES, :] + accb_ref[0:N_NODES, :] + g1_ref[...]
    z = jnp.maximum(dinv * t + b1_ref[...], 0.0)
    g2_ref[...] = dinv * jnp.dot(z, w2_ref[...],
                                 preferred_element_type=jnp.float32)


def _tc3_body(dinv_ref, acca_ref, accb_ref, g2_ref, b2_ref, out_ref):
    dinv = dinv_ref[0:N_NODES, :]
    t = acca_ref[0:N_NODES, :] + accb_ref[0:N_NODES, :] + g2_ref[...]
    out_ref[...] = dinv * t + b2_ref[...]


def _row_spec(shape):
    return pl.BlockSpec((BROW,) + shape[1:], lambda i: (i,) + (0,) * len(shape[1:]))


def _full_spec(shape):
    return pl.BlockSpec(shape, lambda i: (0,) * len(shape))


def _tc1_call(doc, word, lin_W, lin_b, W1, hist):
    return pl.pallas_call(
        _tc1_body,
        out_shape=[jax.ShapeDtypeStruct((N_NODES, D), jnp.float32),
                   jax.ShapeDtypeStruct((NPAD, 1), jnp.float32)],
        in_specs=[pl.BlockSpec(memory_space=pltpu.VMEM)] * 6,
        out_specs=[pl.BlockSpec(memory_space=pltpu.VMEM)] * 2,
    )(doc, word, lin_W, lin_b.reshape(1, D), W1, hist)


def _tc23_call(body, extra_mm, dinv, acca, accb, g, b, *W):
    return pl.pallas_call(
        body,
        out_shape=jax.ShapeDtypeStruct((N_NODES, D), jnp.float32),
        in_specs=[pl.BlockSpec(memory_space=pltpu.VMEM)] * (5 + extra_mm),
        out_specs=pl.BlockSpec(memory_space=pltpu.VMEM),
    )(dinv, acca, accb, g, b.reshape(1, D), *W)


def kernel(doc_features, word_features, edge_index, mode,
           lin_W, lin_b, W1, b1, W2, b2):
    src = edge_index[0].astype(jnp.int32)
    dst = edge_index[1].astype(jnp.int32)

    zerosd = jnp.zeros((NPAD, D), jnp.float32)

    deg_k, scat_k = _sc_kernels()
    hist = deg_k(dst)

    g1, dinv = _tc1_call(doc_features, word_features, lin_W, lin_b, W1, hist)

    acc1a, acc1b = scat_k(src, dst, g1, zerosd)

    g2 = _tc23_call(_tc2_body, 1, dinv, acc1a, acc1b, g1, b1, W2)

    acc2a, acc2b = scat_k(src, dst, g2, zerosd)

    out = _tc23_call(_tc3_body, 0, dinv, acc2a, acc2b, g2, b2)
    return out


# speedup vs baseline: 1.0156x; 1.0156x over previous
import functools

import jax
import jax.numpy as jnp
from jax import lax
from jax.experimental import pallas as pl
from jax.experimental.pallas import tpu as pltpu
from jax.experimental.pallas import tpu_sc as plsc

N_DOC = 2000
N_NODES = 10000
N_EDGES = 320000
D = 128
NPAD = 10240

NC, NS = 2, 16
NW = NC * NS
EPT = N_EDGES // NW
K = 80
NSTEPS = EPT // K
RPT = NPAD // NS
L = 16
CH = 2000
NBUF = 4


@functools.cache
def _sc_kernels():
    mesh = plsc.VectorSubcoreMesh(core_axis_name="c", subcore_axis_name="s",
                                  num_cores=NC, num_subcores=NS)
    deg = functools.partial(
        pl.kernel,
        out_type=jax.ShapeDtypeStruct((NW, NPAD), jnp.float32),
        mesh=mesh,
        compiler_params=pltpu.CompilerParams(needs_layout_passes=False),
        scratch_types=[
            pltpu.VMEM((CH,), jnp.int32),
            pltpu.VMEM((NPAD,), jnp.float32),
        ],
    )(_deg_body)
    scat = functools.partial(
        pl.kernel,
        out_type=jax.ShapeDtypeStruct((NC * NPAD, D), jnp.float32),
        mesh=mesh,
        scratch_types=(
            [pltpu.VMEM((K,), jnp.int32)] * NBUF
            + [pltpu.VMEM((K,), jnp.int32)] * NBUF
            + [pltpu.VMEM((K, D), jnp.float32)] * NBUF
            + [pltpu.VMEM_SHARED((NPAD, D), jnp.float32)]
            + [pltpu.SemaphoreType.DMA] * (3 * NBUF)
        ),
    )(_edge_scatter_body)
    return deg, scat


def _deg_body(dst_hbm, out_hbm, didx_v, hist_v):
    c = lax.axis_index("c")
    s = lax.axis_index("s")
    wid = c * NS + s
    base = wid * EPT
    zero = jnp.zeros((L,), jnp.float32)

    @pl.loop(0, NPAD // L)
    def _(i):
        hist_v[pl.ds(i * L, L)] = zero

    ones = jnp.ones((L,), jnp.float32)

    @pl.loop(0, EPT // CH)
    def _(jc):
        pltpu.sync_copy(dst_hbm.at[pl.ds(base + jc * CH, CH)], didx_v)

        @pl.loop(0, CH // L)
        def _(i):
            idx = didx_v[pl.ds(i * L, L)]
            plsc.addupdate_scatter(hist_v, [idx], ones)

    pltpu.sync_copy(hist_v, out_hbm.at[wid])


def _edge_scatter_body(src_hbm, dst_hbm, g_hbm, zerosd_hbm, out_hbm, *refs):
    sidx = refs[0:NBUF]
    didx = refs[NBUF:2 * NBUF]
    rows = refs[2 * NBUF:3 * NBUF]
    acc_s = refs[3 * NBUF]
    isem = refs[3 * NBUF + 1:4 * NBUF + 1]
    gsem = refs[4 * NBUF + 1:5 * NBUF + 1]
    ssem = refs[5 * NBUF + 1:6 * NBUF + 1]
    c = lax.axis_index("c")
    s = lax.axis_index("s")
    base = (c * NS + s) * EPT
    r0 = s * RPT

    pltpu.sync_copy(zerosd_hbm.at[pl.ds(r0, RPT)], acc_s.at[pl.ds(r0, RPT)])
    plsc.subcore_barrier()

    def ifire(j, b):
        e0 = base + j * K
        pltpu.async_copy(src_hbm.at[pl.ds(e0, K)], sidx[b], isem[b])
        pltpu.async_copy(dst_hbm.at[pl.ds(e0, K)], didx[b], isem[b])

    def iwait(b):
        pltpu.make_async_copy(src_hbm.at[pl.ds(base, K)], sidx[b],
                              isem[b]).wait()
        pltpu.make_async_copy(dst_hbm.at[pl.ds(base, K)], didx[b],
                              isem[b]).wait()

    def gfire(b):
        pltpu.async_copy(g_hbm.at[sidx[b]], rows[b], gsem[b])

    def gwait(b):
        pltpu.make_async_copy(g_hbm.at[sidx[b]], rows[b], gsem[b]).wait()

    def sfire(b):
        pltpu.async_copy(rows[b], acc_s.at[didx[b]], ssem[b], add=True)

    def swait(b):
        pltpu.make_async_copy(rows[b], acc_s.at[didx[b]], ssem[b]).wait()

    def body(j, m, do_swait=True, do_i=True, do_g=True):
        if do_swait:
            swait((m + NBUF - 1) % NBUF)
        if do_i:
            ifire(j + 3, (m + 3) % NBUF)
        if do_g:
            iwait((m + 2) % NBUF)
            gfire((m + 2) % NBUF)
        gwait(m)
        sfire(m)

    ifire(0, 0)
    ifire(1, 1)
    ifire(2, 2)
    iwait(0)
    gfire(0)
    iwait(1)
    gfire(1)
    body(0, 0, do_swait=False)
    body(1, 1)
    body(2, 2)

    NGRP = (NSTEPS - 3 - 6) // NBUF

    @pl.loop(0, NGRP)
    def _(p):
        j = 3 + NBUF * p
        for t in range(NBUF):
            body(j + t, (3 + t) % NBUF)

    for j in range(3 + NBUF * NGRP, NSTEPS):
        body(j, j % NBUF, do_i=(j + 3 < NSTEPS), do_g=(j + 2 < NSTEPS))
    swait((NSTEPS - 1) % NBUF)

    plsc.subcore_barrier()
    pltpu.sync_copy(acc_s.at[pl.ds(r0, RPT)],
                    out_hbm.at[pl.ds(c * NPAD + r0, RPT)])


def _dinv_from_hist(hist):
    deg = jnp.sum(hist, axis=0)[0:N_NODES, None] + 1.0
    return lax.rsqrt(deg)


def _tc1_body(doc_ref, word_ref, linw_ref, linb_ref, w1_ref, hist_ref, g1_ref):
    dinv = _dinv_from_hist(hist_ref[...])
    wf = jnp.dot(word_ref[...], linw_ref[...],
                 preferred_element_type=jnp.float32) + linb_ref[...]
    hd = jnp.dot(doc_ref[...], w1_ref[...], preferred_element_type=jnp.float32)
    hw = jnp.dot(wf, w1_ref[...], preferred_element_type=jnp.float32)
    g1_ref[0:N_DOC, :] = dinv[0:N_DOC] * hd
    g1_ref[N_DOC:N_NODES, :] = dinv[N_DOC:N_NODES] * hw


def _tc2_body(hist_ref, acc_ref, g1_ref, b1_ref, w2_ref, g2_ref):
    dinv = _dinv_from_hist(hist_ref[...])
    t = acc_ref[0:N_NODES, :] + acc_ref[NPAD:NPAD + N_NODES, :] + g1_ref[...]
    z = jnp.maximum(dinv * t + b1_ref[...], 0.0)
    g2_ref[...] = dinv * jnp.dot(z, w2_ref[...],
                                 preferred_element_type=jnp.float32)


def _tc3_body(hist_ref, acc_ref, g2_ref, b2_ref, out_ref):
    dinv = _dinv_from_hist(hist_ref[...])
    t = acc_ref[0:N_NODES, :] + acc_ref[NPAD:NPAD + N_NODES, :] + g2_ref[...]
    out_ref[...] = dinv * t + b2_ref[...]


def _vmem_call(body, n_in, out_shape):
    return pl.pallas_call(
        body,
        out_shape=out_shape,
        in_specs=[pl.BlockSpec(memory_space=pltpu.VMEM)] * n_in,
        out_specs=pl.BlockSpec(memory_space=pltpu.VMEM),
    )


def kernel(doc_features, word_features, edge_index, mode,
           lin_W, lin_b, W1, b1, W2, b2):
    src = edge_index[0].astype(jnp.int32)
    dst = edge_index[1].astype(jnp.int32)

    zerosd = jnp.zeros((NPAD, D), jnp.float32)

    deg_k, scat_k = _sc_kernels()
    hist = deg_k(dst)

    g1 = _vmem_call(_tc1_body, 6,
                    jax.ShapeDtypeStruct((N_NODES, D), jnp.float32))(
        doc_features, word_features, lin_W, lin_b.reshape(1, D), W1, hist)

    acc1 = scat_k(src, dst, g1, zerosd)

    g2 = _vmem_call(_tc2_body, 5,
                    jax.ShapeDtypeStruct((N_NODES, D), jnp.float32))(
        hist, acc1, g1, b1.reshape(1, D), W2)

    acc2 = scat_k(src, dst, g2, zerosd)

    out = _vmem_call(_tc3_body, 4,
                     jax.ShapeDtypeStruct((N_NODES, D), jnp.float32))(
        hist, acc2, g2, b2.reshape(1, D))
    return out
